# baseline (speedup 1.0000x reference)
import functools

import jax
import jax.numpy as jnp
from jax import lax
from jax.experimental import pallas as pl
from jax.experimental.pallas import tpu as pltpu
from jax.experimental.pallas import tpu_sc as plsc


def _base_body(m_ref, p_ref, vw_ref, base_ref):
    base_ref[...] = m_ref[...] + p_ref[...] + vw_ref[...]


def _tc_body(np2, x_ref, wt_ref, b_ref, ids_ref, idc_ref, p_ref,
             vw_ref, v_ref, gid_ref):
    bidx = pl.program_id(0)
    nv = ids_ref.shape[2]
    pe = p_ref[...] + vw_ref[...]
    oh = (idc_ref[0] == lax.broadcasted_iota(jnp.int32, (nv, np2), 1)
          ).astype(jnp.float32)
    pe_g = jnp.dot(oh, pe,
                   preferred_element_type=jnp.float32)
    xd = jnp.dot(x_ref[0], wt_ref[...],
                 preferred_element_type=jnp.float32)
    v_ref[0] = xd + pe_g + b_ref[...]
    gid_ref[0] = ids_ref[0] + bidx * np2


def _sc_out_body(np2, rpw, ch, v_hbm, gid_hbm, base_hbm, out_hbm,
                 idx2, mark, comp_l, comp_g, buf0, buf1,
                 semi0, semi1, semo0, semo1):
    c = lax.axis_index("c")
    s = lax.axis_index("s")
    wid = s * 2 + c
    row0 = wid * rpw
    nv_b = rpw // 2
    nchv = nv_b // ch
    nchc = (np2 - nv_b) // ch
    ntot = nchv + nchc
    bufs = (buf0, buf1)
    semis = (semi0, semi1)
    semos = (semo0, semo1)
    pltpu.sync_copy(gid_hbm.at[wid], idx2)
    zero16 = jnp.zeros((16,), jnp.int32)
    ones16 = jnp.ones((16,), jnp.int32)
    iota16 = lax.iota(jnp.int32, 16)

    def batch_body(t, carry):
        gbase = (2 * wid + t) * np2
        for j in range(np2 // 16):
            mark[pl.ds(j * 16, 16)] = zero16
        for kk in range(nchv):
            for j in range(ch // 16):
                g = idx2[t * nchv + kk, pl.ds(j * 16, 16)]
                plsc.store_scatter(mark, [g - gbase], ones16)

        def comp_body(j, off):
            mv = mark[pl.ds(j * 16, 16)]
            msk = mv == 0
            inc = msk.astype(jnp.int32)
            loc = plsc.cumsum(inc)
            tgt = off + loc - 1
            d1 = tgt // ch
            d2 = tgt - d1 * ch
            pos = iota16 + j * 16
            plsc.store_scatter(comp_l, [d1, d2], pos, mask=msk)
            plsc.store_scatter(comp_g, [d1, d2], pos + gbase, mask=msk)
            return off + jnp.sum(inc)

        lax.fori_loop(0, np2 // 16, comp_body, 0)

        def start_in(i):
            bb, sm = bufs[i % 2], semis[i % 2]
            if i < nchv:
                return pltpu.async_copy(
                    v_hbm.at[pl.ds(row0 + (t * nchv + i) * ch, ch)], bb, sm)
            return pltpu.async_copy(base_hbm.at[comp_l.at[i - nchv]], bb, sm)

        in_cp = [None] * ntot
        out_cp = [None] * ntot
        in_cp[0] = start_in(0)
        for k in range(ntot):
            if k + 1 < ntot:
                if k >= 1:
                    out_cp[k - 1].wait()
                in_cp[k + 1] = start_in(k + 1)
            in_cp[k].wait()
            iref = idx2.at[t * nchv + k] if k < nchv else comp_g.at[k - nchv]
            out_cp[k] = pltpu.async_copy(bufs[k % 2], out_hbm.at[iref],
                                         semos[k % 2])
        out_cp[ntot - 2].wait()
        out_cp[ntot - 1].wait()
        return carry

    lax.fori_loop(0, 2, batch_body, 0)


def kernel(x, visible_ids, W, b, mask_tokens, pos_embeds, view_embeds):
    B, NV, ED = x.shape
    DD = W.shape[0]
    NP2 = mask_tokens.shape[1]

    wt = W.T
    b2 = b.reshape(1, DD)
    ids3 = visible_ids.reshape(B, 1, NV)
    idc3 = visible_ids.reshape(B, NV, 1)
    m2 = mask_tokens.reshape(NP2, DD)
    p2 = pos_embeds.reshape(NP2, DD)
    vw2 = view_embeds.reshape(NP2, DD)

    base = pl.pallas_call(
        _base_body,
        out_shape=jax.ShapeDtypeStruct((NP2, DD), jnp.float32),
    )(m2, p2, vw2)

    v, gid = pl.pallas_call(
        functools.partial(_tc_body, NP2),
        grid=(B,),
        in_specs=[
            pl.BlockSpec((1, NV, ED), lambda i: (i, 0, 0)),
            pl.BlockSpec((ED, DD), lambda i: (0, 0)),
            pl.BlockSpec((1, DD), lambda i: (0, 0)),
            pl.BlockSpec((1, 1, NV), lambda i: (i, 0, 0)),
            pl.BlockSpec((1, NV, 1), lambda i: (i, 0, 0)),
            pl.BlockSpec((NP2, DD), lambda i: (0, 0)),
            pl.BlockSpec((NP2, DD), lambda i: (0, 0)),
        ],
        out_specs=[
            pl.BlockSpec((1, NV, DD), lambda i: (i, 0, 0)),
            pl.BlockSpec((1, 1, NV), lambda i: (i, 0, 0)),
        ],
        out_shape=[
            jax.ShapeDtypeStruct((B, NV, DD), jnp.float32),
            jax.ShapeDtypeStruct((B, 1, NV), jnp.int32),
        ],
        compiler_params=pltpu.CompilerParams(
            dimension_semantics=("arbitrary",)),
    )(x, wt, b2, ids3, idc3, p2, vw2)

    info = plsc.get_sparse_core_info()
    nw = info.num_cores * info.num_subcores
    rows = B * NV
    rpw = rows // nw
    ch = 96
    nch = rpw // ch

    mesh = plsc.VectorSubcoreMesh(core_axis_name="c", subcore_axis_name="s")

    sc_out = functools.partial(
        pl.kernel,
        out_type=jax.ShapeDtypeStruct((B * NP2, DD), jnp.float32),
        mesh=mesh,
        compiler_params=pltpu.CompilerParams(needs_layout_passes=False),
        scratch_types=[
            pltpu.VMEM((nch, ch), jnp.int32),
            pltpu.VMEM((NP2,), jnp.int32),
            pltpu.VMEM(((NP2 - rpw // 2) // ch, ch), jnp.int32),
            pltpu.VMEM(((NP2 - rpw // 2) // ch, ch), jnp.int32),
            pltpu.VMEM((ch, DD), jnp.float32),
            pltpu.VMEM((ch, DD), jnp.float32),
            pltpu.SemaphoreType.DMA,
            pltpu.SemaphoreType.DMA,
            pltpu.SemaphoreType.DMA,
            pltpu.SemaphoreType.DMA,
        ],
    )(functools.partial(_sc_out_body, NP2, rpw, ch))

    out = sc_out(v.reshape(rows, DD), gid.reshape(nw, nch, ch), base)
    return out.reshape(B, NP2, DD)


# speedup vs baseline: 1.4981x; 1.4981x over previous
import functools

import jax
import jax.numpy as jnp
from jax import lax
from jax.experimental import pallas as pl
from jax.experimental.pallas import tpu as pltpu
from jax.experimental.pallas import tpu_sc as plsc


def _base_body(m_ref, p_ref, vw_ref, base_ref):
    base_ref[...] = m_ref[...] + p_ref[...] + vw_ref[...]


def _tc_body(np2, x_ref, wt_ref, b_ref, ids_ref, idc_ref, p_ref,
             vw_ref, v_ref, gid_ref):
    bidx = pl.program_id(0)
    nv = ids_ref.shape[2]
    pe = p_ref[...] + vw_ref[...]
    oh = (idc_ref[0] == lax.broadcasted_iota(jnp.int32, (nv, np2), 1)
          ).astype(jnp.float32)
    pe_g = jnp.dot(oh, pe,
                   preferred_element_type=jnp.float32)
    xd = jnp.dot(x_ref[0], wt_ref[...],
                 preferred_element_type=jnp.float32)
    v_ref[0] = xd + pe_g + b_ref[...]
    gid_ref[0] = ids_ref[0] + bidx * np2


def _sc_fill_body(np2, base_hbm, out_hbm, base_sh, sem):
    c = lax.axis_index("c")
    s = lax.axis_index("s")
    wid = s * 2 + c
    @pl.when(s == 0)
    def _():
        pltpu.async_copy(base_hbm, base_sh, sem).wait()
    plsc.subcore_barrier()
    f0 = pltpu.async_copy(base_sh, out_hbm.at[pl.ds((2 * wid) * np2, np2)],
                          sem)
    f1 = pltpu.async_copy(base_sh,
                          out_hbm.at[pl.ds((2 * wid + 1) * np2, np2)], sem)
    f0.wait()
    f1.wait()


def _sc_scatter_body(rpw, ch, v_hbm, gid_hbm, out_hbm,
                     idx2, buf0, buf1, semi0, semi1, semo0, semo1):
    c = lax.axis_index("c")
    s = lax.axis_index("s")
    wid = s * 2 + c
    base = wid * rpw
    nch = rpw // ch
    bufs = (buf0, buf1)
    semis = (semi0, semi1)
    semos = (semo0, semo1)
    pltpu.sync_copy(gid_hbm.at[wid], idx2)
    in_cp = [None] * nch
    out_cp = [None] * nch
    in_cp[0] = pltpu.async_copy(v_hbm.at[pl.ds(base, ch)], buf0, semi0)
    for k in range(nch):
        if k + 1 < nch:
            if k - 1 >= 0:
                out_cp[k - 1].wait()
            in_cp[k + 1] = pltpu.async_copy(
                v_hbm.at[pl.ds(base + (k + 1) * ch, ch)],
                bufs[(k + 1) % 2], semis[(k + 1) % 2])
        in_cp[k].wait()
        out_cp[k] = pltpu.async_copy(bufs[k % 2], out_hbm.at[idx2.at[k]],
                                     semos[k % 2])
    out_cp[nch - 2].wait()
    out_cp[nch - 1].wait()


def kernel(x, visible_ids, W, b, mask_tokens, pos_embeds, view_embeds):
    B, NV, ED = x.shape
    DD = W.shape[0]
    NP2 = mask_tokens.shape[1]

    wt = W.T
    b2 = b.reshape(1, DD)
    ids3 = visible_ids.reshape(B, 1, NV)
    idc3 = visible_ids.reshape(B, NV, 1)
    m2 = mask_tokens.reshape(NP2, DD)
    p2 = pos_embeds.reshape(NP2, DD)
    vw2 = view_embeds.reshape(NP2, DD)

    base = pl.pallas_call(
        _base_body,
        out_shape=jax.ShapeDtypeStruct((NP2, DD), jnp.float32),
    )(m2, p2, vw2)

    v, gid = pl.pallas_call(
        functools.partial(_tc_body, NP2),
        grid=(B,),
        in_specs=[
            pl.BlockSpec((1, NV, ED), lambda i: (i, 0, 0)),
            pl.BlockSpec((ED, DD), lambda i: (0, 0)),
            pl.BlockSpec((1, DD), lambda i: (0, 0)),
            pl.BlockSpec((1, 1, NV), lambda i: (i, 0, 0)),
            pl.BlockSpec((1, NV, 1), lambda i: (i, 0, 0)),
            pl.BlockSpec((NP2, DD), lambda i: (0, 0)),
            pl.BlockSpec((NP2, DD), lambda i: (0, 0)),
        ],
        out_specs=[
            pl.BlockSpec((1, NV, DD), lambda i: (i, 0, 0)),
            pl.BlockSpec((1, 1, NV), lambda i: (i, 0, 0)),
        ],
        out_shape=[
            jax.ShapeDtypeStruct((B, NV, DD), jnp.float32),
            jax.ShapeDtypeStruct((B, 1, NV), jnp.int32),
        ],
        compiler_params=pltpu.CompilerParams(
            dimension_semantics=("arbitrary",)),
    )(x, wt, b2, ids3, idc3, p2, vw2)

    info = plsc.get_sparse_core_info()
    nw = info.num_cores * info.num_subcores
    rows = B * NV
    rpw = rows // nw
    ch = 96
    nch = rpw // ch

    mesh = plsc.VectorSubcoreMesh(core_axis_name="c", subcore_axis_name="s")

    sc_fill = functools.partial(
        pl.kernel,
        out_type=jax.ShapeDtypeStruct((B * NP2, DD), jnp.float32),
        mesh=mesh,
        scratch_types=[
            pltpu.VMEM_SHARED((NP2, DD), jnp.float32),
            pltpu.SemaphoreType.DMA,
        ],
    )(functools.partial(_sc_fill_body, NP2))

    sc_scatter = functools.partial(
        pl.kernel,
        out_type=(),
        mesh=mesh,
        scratch_types=[
            pltpu.VMEM((nch, ch), jnp.int32),
            pltpu.VMEM((ch, DD), jnp.float32),
            pltpu.VMEM((ch, DD), jnp.float32),
            pltpu.SemaphoreType.DMA,
            pltpu.SemaphoreType.DMA,
            pltpu.SemaphoreType.DMA,
            pltpu.SemaphoreType.DMA,
        ],
    )(functools.partial(_sc_scatter_body, rpw, ch))

    out_fill = sc_fill(base)
    out_ref = jax.new_ref(out_fill)
    sc_scatter(v.reshape(rows, DD), gid.reshape(nw, nch, ch), out_ref)
    return jax.freeze(out_ref).reshape(B, NP2, DD)


# speedup vs baseline: 1.7335x; 1.1572x over previous
import functools

import jax
import jax.numpy as jnp
from jax import lax
from jax.experimental import pallas as pl
from jax.experimental.pallas import tpu as pltpu
from jax.experimental.pallas import tpu_sc as plsc


def _base_body(m_ref, p_ref, vw_ref, base_ref):
    base_ref[...] = m_ref[...] + p_ref[...] + vw_ref[...]


def _tc_body(np2, bb, x_ref, wt_ref, b_ref, ids_ref, idc_ref, p_ref,
             vw_ref, v_ref, gid_ref):
    bidx = pl.program_id(0)
    nv = ids_ref.shape[2]
    ed = x_ref.shape[2]
    dd = wt_ref.shape[1]
    pe = p_ref[...] + vw_ref[...]
    idc = idc_ref[...].reshape(bb * nv, 1)
    oh = (idc == lax.broadcasted_iota(jnp.int32, (bb * nv, np2), 1)
          ).astype(jnp.float32)
    pe_g = jnp.dot(oh, pe,
                   preferred_element_type=jnp.float32)
    xd = jnp.dot(x_ref[...].reshape(bb * nv, ed), wt_ref[...],
                 preferred_element_type=jnp.float32)
    v_ref[...] = (xd + pe_g + b_ref[...]).reshape(bb, nv, dd)
    for t in range(bb):
        gid_ref[t] = ids_ref[t] + (bidx * bb + t) * np2


def _sc_fill_body(np2, base_hbm, out_hbm, base_sh, sem):
    c = lax.axis_index("c")
    s = lax.axis_index("s")
    wid = s * 2 + c
    @pl.when(s == 0)
    def _():
        pltpu.async_copy(base_hbm, base_sh, sem).wait()
    plsc.subcore_barrier()
    f0 = pltpu.async_copy(base_sh, out_hbm.at[pl.ds((2 * wid) * np2, np2)],
                          sem)
    f1 = pltpu.async_copy(base_sh,
                          out_hbm.at[pl.ds((2 * wid + 1) * np2, np2)], sem)
    f0.wait()
    f1.wait()


def _sc_scatter_body(rpw, ch, v_hbm, gid_hbm, out_hbm,
                     idx2, buf0, buf1, semi0, semi1, semo0, semo1):
    c = lax.axis_index("c")
    s = lax.axis_index("s")
    wid = s * 2 + c
    base = wid * rpw
    nch = rpw // ch
    bufs = (buf0, buf1)
    semis = (semi0, semi1)
    semos = (semo0, semo1)
    pltpu.sync_copy(gid_hbm.at[wid], idx2)
    in_cp = [None] * nch
    out_cp = [None] * nch
    in_cp[0] = pltpu.async_copy(v_hbm.at[pl.ds(base, ch)], buf0, semi0)
    for k in range(nch):
        if k + 1 < nch:
            if k - 1 >= 0:
                out_cp[k - 1].wait()
            in_cp[k + 1] = pltpu.async_copy(
                v_hbm.at[pl.ds(base + (k + 1) * ch, ch)],
                bufs[(k + 1) % 2], semis[(k + 1) % 2])
        in_cp[k].wait()
        out_cp[k] = pltpu.async_copy(bufs[k % 2], out_hbm.at[idx2.at[k]],
                                     semos[k % 2])
    out_cp[nch - 2].wait()
    out_cp[nch - 1].wait()


def kernel(x, visible_ids, W, b, mask_tokens, pos_embeds, view_embeds):
    B, NV, ED = x.shape
    DD = W.shape[0]
    NP2 = mask_tokens.shape[1]

    wt = W.T
    b2 = b.reshape(1, DD)
    ids3 = visible_ids.reshape(B, 1, NV)
    idc3 = visible_ids.reshape(B, NV, 1)
    m2 = mask_tokens.reshape(NP2, DD)
    p2 = pos_embeds.reshape(NP2, DD)
    vw2 = view_embeds.reshape(NP2, DD)

    base = pl.pallas_call(
        _base_body,
        out_shape=jax.ShapeDtypeStruct((NP2, DD), jnp.float32),
    )(m2, p2, vw2)

    BB = 2
    v, gid = pl.pallas_call(
        functools.partial(_tc_body, NP2, BB),
        grid=(B // BB,),
        in_specs=[
            pl.BlockSpec((BB, NV, ED), lambda i: (i, 0, 0)),
            pl.BlockSpec((ED, DD), lambda i: (0, 0)),
            pl.BlockSpec((1, DD), lambda i: (0, 0)),
            pl.BlockSpec((BB, 1, NV), lambda i: (i, 0, 0)),
            pl.BlockSpec((BB, NV, 1), lambda i: (i, 0, 0)),
            pl.BlockSpec((NP2, DD), lambda i: (0, 0)),
            pl.BlockSpec((NP2, DD), lambda i: (0, 0)),
        ],
        out_specs=[
            pl.BlockSpec((BB, NV, DD), lambda i: (i, 0, 0)),
            pl.BlockSpec((BB, 1, NV), lambda i: (i, 0, 0)),
        ],
        out_shape=[
            jax.ShapeDtypeStruct((B, NV, DD), jnp.float32),
            jax.ShapeDtypeStruct((B, 1, NV), jnp.int32),
        ],
        compiler_params=pltpu.CompilerParams(
            dimension_semantics=("arbitrary",)),
    )(x, wt, b2, ids3, idc3, p2, vw2)

    info = plsc.get_sparse_core_info()
    nw = info.num_cores * info.num_subcores
    rows = B * NV
    rpw = rows // nw
    ch = 96
    nch = rpw // ch

    mesh = plsc.VectorSubcoreMesh(core_axis_name="c", subcore_axis_name="s")

    sc_fill = functools.partial(
        pl.kernel,
        out_type=jax.ShapeDtypeStruct((B * NP2, DD), jnp.float32),
        mesh=mesh,
        scratch_types=[
            pltpu.VMEM_SHARED((NP2, DD), jnp.float32),
            pltpu.SemaphoreType.DMA,
        ],
    )(functools.partial(_sc_fill_body, NP2))

    sc_scatter = functools.partial(
        pl.kernel,
        out_type=(),
        mesh=mesh,
        scratch_types=[
            pltpu.VMEM((nch, ch), jnp.int32),
            pltpu.VMEM((ch, DD), jnp.float32),
            pltpu.VMEM((ch, DD), jnp.float32),
            pltpu.SemaphoreType.DMA,
            pltpu.SemaphoreType.DMA,
            pltpu.SemaphoreType.DMA,
            pltpu.SemaphoreType.DMA,
        ],
    )(functools.partial(_sc_scatter_body, rpw, ch))

    out_fill = sc_fill(base)
    out_ref = jax.new_ref(out_fill)
    sc_scatter(v.reshape(rows, DD), gid.reshape(nw, nch, ch), out_ref)
    return jax.freeze(out_ref).reshape(B, NP2, DD)


# speedup vs baseline: 1.8626x; 1.0745x over previous
import functools

import jax
import jax.numpy as jnp
from jax import lax
from jax.experimental import pallas as pl
from jax.experimental.pallas import tpu as pltpu
from jax.experimental.pallas import tpu_sc as plsc


def _base_body(m_ref, p_ref, vw_ref, base_ref):
    base_ref[...] = m_ref[...] + p_ref[...] + vw_ref[...]


def _tc_body(np2, bb, x_ref, wt_ref, b_ref, ids_ref, idc_ref, p_ref,
             vw_ref, v_ref, gid_ref):
    bidx = pl.program_id(0)
    nv = ids_ref.shape[2]
    ed = x_ref.shape[2]
    dd = wt_ref.shape[1]
    pe = p_ref[...] + vw_ref[...]
    idc = idc_ref[...].reshape(bb * nv, 1)
    oh = (idc == lax.broadcasted_iota(jnp.int32, (bb * nv, np2), 1)
          ).astype(jnp.float32)
    pe_g = jnp.dot(oh, pe,
                   preferred_element_type=jnp.float32)
    xd = jnp.dot(x_ref[...].reshape(bb * nv, ed), wt_ref[...],
                 preferred_element_type=jnp.float32)
    v_ref[...] = (xd + pe_g + b_ref[...]).reshape(bb, nv, dd)
    for t in range(bb):
        gid_ref[t] = ids_ref[t] + (bidx * bb + t) * np2


def _sc_fill_body(np2, base_hbm, out_hbm, base_sh, sem):
    c = lax.axis_index("c")
    s = lax.axis_index("s")
    wid = s * 2 + c
    @pl.when(s == 0)
    def _():
        pltpu.async_copy(base_hbm, base_sh, sem).wait()
    plsc.subcore_barrier()
    f0 = pltpu.async_copy(base_sh, out_hbm.at[pl.ds((2 * wid) * np2, np2)],
                          sem)
    f1 = pltpu.async_copy(base_sh,
                          out_hbm.at[pl.ds((2 * wid + 1) * np2, np2)], sem)
    f0.wait()
    f1.wait()


def _sc_scatter_body(rpw, ch, v_hbm, gid_hbm, out_hbm,
                     idx2, buf0, buf1, semi0, semi1, semo0, semo1):
    c = lax.axis_index("c")
    s = lax.axis_index("s")
    wid = s * 2 + c
    base = wid * rpw
    nch = rpw // ch
    bufs = (buf0, buf1)
    semis = (semi0, semi1)
    semos = (semo0, semo1)
    pltpu.sync_copy(gid_hbm.at[wid], idx2)
    in_cp = [None] * nch
    out_cp = [None] * nch
    in_cp[0] = pltpu.async_copy(v_hbm.at[pl.ds(base, ch)], buf0, semi0)
    for k in range(nch):
        if k + 1 < nch:
            if k - 1 >= 0:
                out_cp[k - 1].wait()
            in_cp[k + 1] = pltpu.async_copy(
                v_hbm.at[pl.ds(base + (k + 1) * ch, ch)],
                bufs[(k + 1) % 2], semis[(k + 1) % 2])
        in_cp[k].wait()
        out_cp[k] = pltpu.async_copy(bufs[k % 2], out_hbm.at[idx2.at[k]],
                                     semos[k % 2])
    out_cp[nch - 2].wait()
    out_cp[nch - 1].wait()


def kernel(x, visible_ids, W, b, mask_tokens, pos_embeds, view_embeds):
    B, NV, ED = x.shape
    DD = W.shape[0]
    NP2 = mask_tokens.shape[1]

    wt = W.T
    b2 = b.reshape(1, DD)
    ids3 = visible_ids.reshape(B, 1, NV)
    idc3 = visible_ids.reshape(B, NV, 1)
    m2 = mask_tokens.reshape(NP2, DD)
    p2 = pos_embeds.reshape(NP2, DD)
    vw2 = view_embeds.reshape(NP2, DD)

    base = pl.pallas_call(
        _base_body,
        out_shape=jax.ShapeDtypeStruct((NP2, DD), jnp.float32),
    )(m2, p2, vw2)

    BB = 4
    v, gid = pl.pallas_call(
        functools.partial(_tc_body, NP2, BB),
        grid=(B // BB,),
        in_specs=[
            pl.BlockSpec((BB, NV, ED), lambda i: (i, 0, 0)),
            pl.BlockSpec((ED, DD), lambda i: (0, 0)),
            pl.BlockSpec((1, DD), lambda i: (0, 0)),
            pl.BlockSpec((BB, 1, NV), lambda i: (i, 0, 0)),
            pl.BlockSpec((BB, NV, 1), lambda i: (i, 0, 0)),
            pl.BlockSpec((NP2, DD), lambda i: (0, 0)),
            pl.BlockSpec((NP2, DD), lambda i: (0, 0)),
        ],
        out_specs=[
            pl.BlockSpec((BB, NV, DD), lambda i: (i, 0, 0)),
            pl.BlockSpec((BB, 1, NV), lambda i: (i, 0, 0)),
        ],
        out_shape=[
            jax.ShapeDtypeStruct((B, NV, DD), jnp.float32),
            jax.ShapeDtypeStruct((B, 1, NV), jnp.int32),
        ],
        compiler_params=pltpu.CompilerParams(
            dimension_semantics=("arbitrary",)),
    )(x, wt, b2, ids3, idc3, p2, vw2)

    info = plsc.get_sparse_core_info()
    nw = info.num_cores * info.num_subcores
    rows = B * NV
    rpw = rows // nw
    ch = 96
    nch = rpw // ch

    mesh = plsc.VectorSubcoreMesh(core_axis_name="c", subcore_axis_name="s")

    sc_fill = functools.partial(
        pl.kernel,
        out_type=jax.ShapeDtypeStruct((B * NP2, DD), jnp.float32),
        mesh=mesh,
        scratch_types=[
            pltpu.VMEM_SHARED((NP2, DD), jnp.float32),
            pltpu.SemaphoreType.DMA,
        ],
    )(functools.partial(_sc_fill_body, NP2))

    sc_scatter = functools.partial(
        pl.kernel,
        out_type=(),
        mesh=mesh,
        scratch_types=[
            pltpu.VMEM((nch, ch), jnp.int32),
            pltpu.VMEM((ch, DD), jnp.float32),
            pltpu.VMEM((ch, DD), jnp.float32),
            pltpu.SemaphoreType.DMA,
            pltpu.SemaphoreType.DMA,
            pltpu.SemaphoreType.DMA,
            pltpu.SemaphoreType.DMA,
        ],
    )(functools.partial(_sc_scatter_body, rpw, ch))

    out_fill = sc_fill(base)
    out_ref = jax.new_ref(out_fill)
    sc_scatter(v.reshape(rows, DD), gid.reshape(nw, nch, ch), out_ref)
    return jax.freeze(out_ref).reshape(B, NP2, DD)
